# baseline (speedup 1.0000x reference)
import functools

import jax
import jax.numpy as jnp
from jax import lax
from jax.experimental import pallas as pl
from jax.experimental.pallas import tpu as pltpu
from jax.experimental.pallas import tpu_sc as plsc

_NC = 2
_NS = 16
_NW = _NC * _NS
_B = 16384
_D = 512
_BW = _B // _NW
_C = 32
_NCHUNK = _BW // _C
_L = 16
_WPAD = 2 * _D + _L


def _lane_shuffle(v, idx):
  dnums = lax.GatherDimensionNumbers(
      offset_dims=(), collapsed_slice_dims=(0,), start_index_map=(0,))
  return lax.gather(v, idx[:, None], dnums, slice_sizes=(1,),
                    mode=lax.GatherScatterMode.PROMISE_IN_BOUNDS)


def _hsum16(vecs, lane):
  s = 1
  while len(vecs) > 1:
    nxt = []
    for k in range(0, len(vecs), 2):
      u, w = vecs[k], vecs[k + 1]
      m = (lane & s) == 0
      a = jnp.where(m, u, w)
      b = jnp.where(m, w, u)
      nxt.append(a + _lane_shuffle(b, lane ^ s))
    vecs = nxt
    s *= 2
  return vecs[0]


def _dot_chunk(rows_v, w_v, b_vec, out_v, out_base):
  lane = lax.iota(jnp.int32, _L)
  zero = jnp.zeros((_L,), jnp.float32)

  def g_body(g, carry):
    @plsc.parallel_loop(0, _D // _L, unroll=4,
                        carry=tuple(zero for _ in range(_L)))
    def accs(j, acc):
      joff = pl.multiple_of(j * _L, _L)
      w0 = w_v[pl.ds(joff, _L)]
      w1 = w_v[pl.ds(pl.multiple_of(_D + joff, _L), _L)]
      out = []
      for rr in range(_L):
        r2 = 2 * (g * _L + rr)
        out.append(acc[rr]
                   + rows_v[r2, pl.ds(joff, _L)] * w0
                   + rows_v[r2 + 1, pl.ds(joff, _L)] * w1)
      return tuple(out)

    out_vec = _hsum16(list(accs), lane) + b_vec
    idx = pl.multiple_of(out_base + g * _L, _L)
    out_v[pl.ds(idx, _L)] = out_vec
    return carry

  lax.fori_loop(0, _C // _L, g_body, 0)


def _make_sc_kernel():
  mesh = plsc.VectorSubcoreMesh(core_axis_name="c", subcore_axis_name="s")

  @functools.partial(
      pl.kernel,
      mesh=mesh,
      out_type=jax.ShapeDtypeStruct((_B,), jnp.float32),
      scratch_types=[
          pltpu.VMEM((_WPAD,), jnp.float32),
          pltpu.VMEM((2 * _BW,), jnp.int32),
          pltpu.VMEM((2 * _C, _D), jnp.float32),
          pltpu.VMEM((2 * _C, _D), jnp.float32),
          pltpu.VMEM((_BW,), jnp.float32),
          pltpu.SemaphoreType.DMA,
          pltpu.SemaphoreType.DMA,
      ],
  )
  def sc_kernel(table_hbm, idxf_hbm, wb_hbm, out_hbm,
                w_v, idx_v, rbA, rbB, out_v, semA, semB):
    wid = lax.axis_index("s") * _NC + lax.axis_index("c")
    base = pl.multiple_of(wid * _BW, _BW)
    pltpu.sync_copy(wb_hbm, w_v)
    pltpu.sync_copy(idxf_hbm.at[pl.ds(2 * base, 2 * _BW)], idx_v)
    b_vec = w_v[pl.ds(2 * _D, _L)]

    def gather(c, rb, sem):
      off = pl.multiple_of(c * 2 * _C, 2 * _C)
      return pltpu.make_async_copy(
          table_hbm.at[idx_v.at[pl.ds(off, 2 * _C)]], rb, sem)

    def wait_compute(c, rb, sem):
      gather(c, rb, sem).wait()
      _dot_chunk(rb, w_v, b_vec, out_v, c * _C)

    gather(0, rbA, semA).start()
    gather(1, rbB, semB).start()

    def pair_body(p, carry):
      cA = 2 * p
      wait_compute(cA, rbA, semA)

      @pl.when(p < _NCHUNK // 2 - 1)
      def _():
        gather(cA + 2, rbA, semA).start()

      wait_compute(cA + 1, rbB, semB)

      @pl.when(p < _NCHUNK // 2 - 1)
      def _():
        gather(cA + 3, rbB, semB).start()

      return carry

    lax.fori_loop(0, _NCHUNK // 2, pair_body, 0)
    pltpu.sync_copy(out_v, out_hbm.at[pl.ds(base, _BW)])

  return sc_kernel


_sc_kernel = _make_sc_kernel()


@jax.jit
def kernel(x, table, W, b):
  idxf = x.reshape(-1).astype(jnp.int32)
  wb = jnp.concatenate(
      [W.reshape(-1).astype(jnp.float32),
       jnp.broadcast_to(b.astype(jnp.float32), (_L,))])
  out = _sc_kernel(table, idxf, wb)
  return out.reshape(_B, 1)


# speedup vs baseline: 1.1981x; 1.1981x over previous
import functools

import jax
import jax.numpy as jnp
from jax import lax
from jax.experimental import pallas as pl
from jax.experimental.pallas import tpu as pltpu
from jax.experimental.pallas import tpu_sc as plsc

_NC = 2
_NS = 16
_NW = _NC * _NS
_B = 16384
_D = 512
_BW = _B // _NW
_C = 32
_NCHUNK = _BW // _C
_L = 16
_WPAD = 2 * _D + _L


def _lane_shuffle(v, idx):
  dnums = lax.GatherDimensionNumbers(
      offset_dims=(), collapsed_slice_dims=(0,), start_index_map=(0,))
  return lax.gather(v, idx[:, None], dnums, slice_sizes=(1,),
                    mode=lax.GatherScatterMode.PROMISE_IN_BOUNDS)


def _hsum16(vecs, lane):
  s = 1
  while len(vecs) > 1:
    nxt = []
    for k in range(0, len(vecs), 2):
      u, w = vecs[k], vecs[k + 1]
      m = (lane & s) == 0
      a = jnp.where(m, u, w)
      b = jnp.where(m, w, u)
      nxt.append(a + _lane_shuffle(b, lane ^ s))
    vecs = nxt
    s *= 2
  return vecs[0]


def _dot_chunk(rb0, rb1, w_v, b_vec, out_v, out_base):
  lane = lax.iota(jnp.int32, _L)
  zero = jnp.zeros((_L,), jnp.float32)

  def g_body(g, carry):
    @plsc.parallel_loop(0, _D // _L, unroll=4,
                        carry=tuple(zero for _ in range(_L)))
    def accs(j, acc):
      joff = pl.multiple_of(j * _L, _L)
      w0 = w_v[pl.ds(joff, _L)]
      w1 = w_v[pl.ds(pl.multiple_of(_D + joff, _L), _L)]
      out = []
      for rr in range(_L):
        r = g * _L + rr
        out.append(acc[rr]
                   + rb0[r, pl.ds(joff, _L)] * w0
                   + rb1[r, pl.ds(joff, _L)] * w1)
      return tuple(out)

    out_vec = _hsum16(list(accs), lane) + b_vec
    idx = pl.multiple_of(out_base + g * _L, _L)
    out_v[pl.ds(idx, _L)] = out_vec
    return carry

  lax.fori_loop(0, _C // _L, g_body, 0)


def _make_sc_kernel():
  mesh = plsc.VectorSubcoreMesh(core_axis_name="c", subcore_axis_name="s")

  @functools.partial(
      pl.kernel,
      mesh=mesh,
      out_type=jax.ShapeDtypeStruct((_B,), jnp.float32),
      scratch_types=[
          pltpu.VMEM((_WPAD,), jnp.float32),
          pltpu.VMEM((_BW,), jnp.int32),
          pltpu.VMEM((_BW,), jnp.int32),
          pltpu.VMEM((_C, _D), jnp.float32),
          pltpu.VMEM((_C, _D), jnp.float32),
          pltpu.VMEM((_C, _D), jnp.float32),
          pltpu.VMEM((_C, _D), jnp.float32),
          pltpu.VMEM((_BW,), jnp.float32),
          pltpu.SemaphoreType.DMA,
          pltpu.SemaphoreType.DMA,
          pltpu.SemaphoreType.DMA,
          pltpu.SemaphoreType.DMA,
      ],
  )
  def sc_kernel(table_hbm, idx0_hbm, idx1_hbm, wb_hbm, out_hbm,
                w_v, idx0_v, idx1_v, rbA0, rbA1, rbB0, rbB1, out_v,
                semA0, semA1, semB0, semB1):
    wid = lax.axis_index("s") * _NC + lax.axis_index("c")
    base = pl.multiple_of(wid * _BW, _BW)
    pltpu.sync_copy(wb_hbm, w_v)
    pltpu.sync_copy(idx0_hbm.at[pl.ds(base, _BW)], idx0_v)
    pltpu.sync_copy(idx1_hbm.at[pl.ds(base, _BW)], idx1_v)
    b_vec = w_v[pl.ds(2 * _D, _L)]

    def gathers(c, rb0, rb1, sem0, sem1):
      off = pl.multiple_of(c * _C, _C)
      cp0 = pltpu.make_async_copy(
          table_hbm.at[idx0_v.at[pl.ds(off, _C)]], rb0, sem0)
      cp1 = pltpu.make_async_copy(
          table_hbm.at[idx1_v.at[pl.ds(off, _C)]], rb1, sem1)
      return cp0, cp1

    def start(c, rb0, rb1, sem0, sem1):
      cp0, cp1 = gathers(c, rb0, rb1, sem0, sem1)
      cp0.start()
      cp1.start()

    def wait_compute(c, rb0, rb1, sem0, sem1):
      cp0, cp1 = gathers(c, rb0, rb1, sem0, sem1)
      cp0.wait()
      cp1.wait()
      _dot_chunk(rb0, rb1, w_v, b_vec, out_v, c * _C)

    start(0, rbA0, rbA1, semA0, semA1)
    start(1, rbB0, rbB1, semB0, semB1)

    def pair_body(p, carry):
      cA = 2 * p
      wait_compute(cA, rbA0, rbA1, semA0, semA1)

      @pl.when(p < _NCHUNK // 2 - 1)
      def _():
        start(cA + 2, rbA0, rbA1, semA0, semA1)

      wait_compute(cA + 1, rbB0, rbB1, semB0, semB1)

      @pl.when(p < _NCHUNK // 2 - 1)
      def _():
        start(cA + 3, rbB0, rbB1, semB0, semB1)

      return carry

    lax.fori_loop(0, _NCHUNK // 2, pair_body, 0)
    pltpu.sync_copy(out_v, out_hbm.at[pl.ds(base, _BW)])

  return sc_kernel


_sc_kernel = _make_sc_kernel()


@jax.jit
def kernel(x, table, W, b):
  idx0 = x[:, 0].astype(jnp.int32)
  idx1 = x[:, 1].astype(jnp.int32)
  wb = jnp.concatenate(
      [W.reshape(-1).astype(jnp.float32),
       jnp.broadcast_to(b.astype(jnp.float32), (_L,))])
  out = _sc_kernel(table, idx0, idx1, wb)
  return out.reshape(_B, 1)


# speedup vs baseline: 1.2077x; 1.0080x over previous
import functools

import jax
import jax.numpy as jnp
from jax import lax
from jax.experimental import pallas as pl
from jax.experimental.pallas import tpu as pltpu
from jax.experimental.pallas import tpu_sc as plsc

_NC = 2
_NS = 16
_NW = _NC * _NS
_B = 16384
_D = 512
_BW = _B // _NW
_C = 32
_NCHUNK = _BW // _C
_L = 16
_WPAD = 2 * _D + _L


def _lane_shuffle(v, idx):
  dnums = lax.GatherDimensionNumbers(
      offset_dims=(), collapsed_slice_dims=(0,), start_index_map=(0,))
  return lax.gather(v, idx[:, None], dnums, slice_sizes=(1,),
                    mode=lax.GatherScatterMode.PROMISE_IN_BOUNDS)


def _hsum16(vecs, lane):
  s = 1
  while len(vecs) > 1:
    nxt = []
    for k in range(0, len(vecs), 2):
      u, w = vecs[k], vecs[k + 1]
      m = (lane & s) == 0
      a = jnp.where(m, u, w)
      b = jnp.where(m, w, u)
      nxt.append(a + _lane_shuffle(b, lane ^ s))
    vecs = nxt
    s *= 2
  return vecs[0]


def _dot_chunk(rb0, rb1, w_v, b_vec, out_v, out_base):
  lane = lax.iota(jnp.int32, _L)
  zero = jnp.zeros((_L,), jnp.float32)

  def g_body(g, carry):
    @plsc.parallel_loop(0, _D // _L, unroll=8,
                        carry=tuple(zero for _ in range(_L)))
    def accs(j, acc):
      joff = pl.multiple_of(j * _L, _L)
      w0 = w_v[pl.ds(joff, _L)]
      w1 = w_v[pl.ds(pl.multiple_of(_D + joff, _L), _L)]
      out = []
      for rr in range(_L):
        r = g * _L + rr
        out.append(acc[rr]
                   + rb0[r, pl.ds(joff, _L)] * w0
                   + rb1[r, pl.ds(joff, _L)] * w1)
      return tuple(out)

    out_vec = _hsum16(list(accs), lane) + b_vec
    idx = pl.multiple_of(out_base + g * _L, _L)
    out_v[pl.ds(idx, _L)] = out_vec
    return carry

  lax.fori_loop(0, _C // _L, g_body, 0)


def _make_sc_kernel():
  mesh = plsc.VectorSubcoreMesh(core_axis_name="c", subcore_axis_name="s")

  @functools.partial(
      pl.kernel,
      mesh=mesh,
      out_type=jax.ShapeDtypeStruct((_B,), jnp.float32),
      scratch_types=[
          pltpu.VMEM((_WPAD,), jnp.float32),
          pltpu.VMEM((_BW,), jnp.int32),
          pltpu.VMEM((_BW,), jnp.int32),
          pltpu.VMEM((_C, _D), jnp.float32),
          pltpu.VMEM((_C, _D), jnp.float32),
          pltpu.VMEM((_C, _D), jnp.float32),
          pltpu.VMEM((_C, _D), jnp.float32),
          pltpu.VMEM((_BW,), jnp.float32),
          pltpu.SemaphoreType.DMA,
          pltpu.SemaphoreType.DMA,
          pltpu.SemaphoreType.DMA,
          pltpu.SemaphoreType.DMA,
      ],
  )
  def sc_kernel(table_hbm, idx0_hbm, idx1_hbm, wb_hbm, out_hbm,
                w_v, idx0_v, idx1_v, rbA0, rbA1, rbB0, rbB1, out_v,
                semA0, semA1, semB0, semB1):
    wid = lax.axis_index("s") * _NC + lax.axis_index("c")
    base = pl.multiple_of(wid * _BW, _BW)
    cpw = pltpu.make_async_copy(wb_hbm, w_v, semA0)
    cpi0 = pltpu.make_async_copy(idx0_hbm.at[pl.ds(base, _BW)], idx0_v, semA1)
    cpi1 = pltpu.make_async_copy(idx1_hbm.at[pl.ds(base, _BW)], idx1_v, semB0)
    cpw.start()
    cpi0.start()
    cpi1.start()
    cpw.wait()
    cpi0.wait()
    cpi1.wait()
    b_vec = w_v[pl.ds(2 * _D, _L)]

    def gathers(c, rb0, rb1, sem0, sem1):
      off = pl.multiple_of(c * _C, _C)
      cp0 = pltpu.make_async_copy(
          table_hbm.at[idx0_v.at[pl.ds(off, _C)]], rb0, sem0)
      cp1 = pltpu.make_async_copy(
          table_hbm.at[idx1_v.at[pl.ds(off, _C)]], rb1, sem1)
      return cp0, cp1

    def start(c, rb0, rb1, sem0, sem1):
      cp0, cp1 = gathers(c, rb0, rb1, sem0, sem1)
      cp0.start()
      cp1.start()

    def wait_compute(c, rb0, rb1, sem0, sem1):
      cp0, cp1 = gathers(c, rb0, rb1, sem0, sem1)
      cp0.wait()
      cp1.wait()
      _dot_chunk(rb0, rb1, w_v, b_vec, out_v, c * _C)

    start(0, rbA0, rbA1, semA0, semA1)
    start(1, rbB0, rbB1, semB0, semB1)

    def pair_body(p, carry):
      cA = 2 * p
      wait_compute(cA, rbA0, rbA1, semA0, semA1)

      @pl.when(p < _NCHUNK // 2 - 1)
      def _():
        start(cA + 2, rbA0, rbA1, semA0, semA1)

      wait_compute(cA + 1, rbB0, rbB1, semB0, semB1)

      @pl.when(p < _NCHUNK // 2 - 1)
      def _():
        start(cA + 3, rbB0, rbB1, semB0, semB1)

      return carry

    lax.fori_loop(0, _NCHUNK // 2, pair_body, 0)
    pltpu.sync_copy(out_v, out_hbm.at[pl.ds(base, _BW)])

  return sc_kernel


_sc_kernel = _make_sc_kernel()


@jax.jit
def kernel(x, table, W, b):
  idx0 = x[:, 0].astype(jnp.int32)
  idx1 = x[:, 1].astype(jnp.int32)
  wb = jnp.concatenate(
      [W.reshape(-1).astype(jnp.float32),
       jnp.broadcast_to(b.astype(jnp.float32), (_L,))])
  out = _sc_kernel(table, idx0, idx1, wb)
  return out.reshape(_B, 1)
